# baseline (speedup 1.0000x reference)
import functools

import jax
import jax.numpy as jnp
from jax import lax
from jax.experimental import pallas as pl
from jax.experimental.pallas import tpu as pltpu
from jax.experimental.pallas import tpu_sc as plsc

B = 4096
L = 50
D = 64
V1 = 100001

NC = 2
NS = 16
NW = NC * NS
RPW = B // NW
PAIRS = RPW // 2
NB = 4
GU = 2 * L


KB = 512
NKB = -(-V1 // KB)
ROWS_PAD = NKB * KB


def _tc_body(in_ref, out_ref):
    x = in_ref[...]
    xt = x.T
    buf = jnp.concatenate(
        [xt, jnp.zeros((KB, 128 - D), jnp.float32)], axis=1)
    out_ref[...] = buf.reshape(KB * 128)


_tc_relayout = pl.pallas_call(
    _tc_body,
    grid=(NKB,),
    in_specs=[pl.BlockSpec((D, KB), lambda i: (0, i))],
    out_specs=pl.BlockSpec((KB * 128,), lambda i: (i,)),
    out_shape=jax.ShapeDtypeStruct((ROWS_PAD * 128,), jnp.float32),
)


def _build():
    mesh = plsc.VectorSubcoreMesh(core_axis_name="c", subcore_axis_name="s")

    @functools.partial(
        pl.kernel,
        out_type=jax.ShapeDtypeStruct((B, D), jnp.float32),
        mesh=mesh,
        compiler_params=pltpu.CompilerParams(use_tc_tiling_on_sc=False),
        scratch_types=[
            pltpu.VMEM((PAIRS, GU), jnp.int32),
            pltpu.VMEM((RPW,), jnp.int32),
            pltpu.VMEM((RPW + 16,), jnp.float32),
            pltpu.VMEM((RPW, D), jnp.float32),
        ] + [pltpu.VMEM((GU, D), jnp.float32)] * NB
          + [pltpu.SemaphoreType.DMA] * NB,
    )
    def k(kw_h, len_h, table_h, out_h, idx_v, len_v, recip_v, out_v, *rs):
        rbs, sems = rs[:NB], rs[NB:]
        wid = lax.axis_index("s") * NC + lax.axis_index("c")
        row_base = wid * RPW
        pair_base = wid * PAIRS

        pltpu.sync_copy(kw_h.at[pl.ds(pair_base, PAIRS)], idx_v)
        pltpu.sync_copy(len_h.at[pl.ds(row_base, RPW)], len_v)
        for g in range(RPW // 16):
            lv = len_v[pl.ds(g * 16, 16)]
            recip_v[pl.ds(g * 16, 16)] = 1.0 / jnp.maximum(lv, 1).astype(
                jnp.float32)

        def start(p, rb, sem):
            pltpu.async_copy(table_h.at[idx_v.at[p]], rb, sem)

        def wait(p, rb, sem):
            pltpu.make_async_copy(table_h.at[idx_v.at[p]], rb, sem).wait()

        def process(p, rb):
            def lbody(l, accs):
                a0, a1, a2, a3, b0, b1, b2, b3 = accs
                return (
                    a0 + rb[l, pl.ds(0, 16)],
                    a1 + rb[l, pl.ds(16, 16)],
                    a2 + rb[l, pl.ds(32, 16)],
                    a3 + rb[l, pl.ds(48, 16)],
                    b0 + rb[l + L, pl.ds(0, 16)],
                    b1 + rb[l + L, pl.ds(16, 16)],
                    b2 + rb[l + L, pl.ds(32, 16)],
                    b3 + rb[l + L, pl.ds(48, 16)],
                )

            z = jnp.zeros((16,), jnp.float32)
            accs = lax.fori_loop(0, L, lbody, (z, z, z, z, z, z, z, z),
                                 unroll=10)
            j0 = 2 * p
            j1 = j0 + 1
            sv = recip_v[pl.ds(j0, 16)]
            s0 = sv[0]
            s1 = sv[1]
            out_v[j0, pl.ds(0, 16)] = accs[0] * s0
            out_v[j0, pl.ds(16, 16)] = accs[1] * s0
            out_v[j0, pl.ds(32, 16)] = accs[2] * s0
            out_v[j0, pl.ds(48, 16)] = accs[3] * s0
            out_v[j1, pl.ds(0, 16)] = accs[4] * s1
            out_v[j1, pl.ds(16, 16)] = accs[5] * s1
            out_v[j1, pl.ds(32, 16)] = accs[6] * s1
            out_v[j1, pl.ds(48, 16)] = accs[7] * s1

        for b in range(NB):
            start(b, rbs[b], sems[b])

        def step(s, carry):
            p0 = NB * s
            for b in range(NB):
                wait(p0 + b, rbs[b], sems[b])
                process(p0 + b, rbs[b])
                start(p0 + b + NB, rbs[b], sems[b])
            return carry

        lax.fori_loop(0, PAIRS // NB - 1, step, 0)
        for b in range(NB):
            p = PAIRS - NB + b
            wait(p, rbs[b], sems[b])
            process(p, rbs[b])

        pltpu.sync_copy(out_v, out_h.at[pl.ds(row_base, RPW)])

    return k


_sc_kernel = _build()


def kernel(keyword_lists, keyword_lengths, table):
    kw = (keyword_lists * 2).reshape(NW * PAIRS, GU)
    lens = keyword_lengths.reshape(B)
    tlin = _tc_relayout(table.T).reshape(2 * ROWS_PAD, D)
    return _sc_kernel(kw, lens, tlin)


# speedup vs baseline: 1.6906x; 1.6906x over previous
import functools

import jax
import jax.numpy as jnp
from jax import lax
from jax.experimental import pallas as pl
from jax.experimental.pallas import tpu as pltpu
from jax.experimental.pallas import tpu_sc as plsc

B = 4096
L = 50
D = 64
V1 = 100001

NC = 2
NS = 16
NW = NC * NS
RPW = B // NW
PAIRS = RPW // 2
NB = 4
GU = 2 * L


def _build():
    mesh = plsc.VectorSubcoreMesh(core_axis_name="c", subcore_axis_name="s")

    @functools.partial(
        pl.kernel,
        out_type=jax.ShapeDtypeStruct((B, D), jnp.float32),
        mesh=mesh,
        compiler_params=pltpu.CompilerParams(use_tc_tiling_on_sc=False),
        scratch_types=[
            pltpu.VMEM((PAIRS, GU), jnp.int32),
            pltpu.VMEM((RPW,), jnp.int32),
            pltpu.VMEM((RPW + 16,), jnp.float32),
            pltpu.VMEM((RPW, D), jnp.float32),
        ] + [pltpu.VMEM((GU, D), jnp.float32)] * NB
          + [pltpu.SemaphoreType.DMA] * NB,
    )
    def k(kw_h, len_h, table_h, out_h, idx_v, len_v, recip_v, out_v, *rs):
        rbs, sems = rs[:NB], rs[NB:]
        wid = lax.axis_index("s") * NC + lax.axis_index("c")
        row_base = wid * RPW
        pair_base = wid * PAIRS

        pltpu.sync_copy(kw_h.at[pl.ds(pair_base, PAIRS)], idx_v)
        pltpu.sync_copy(len_h.at[pl.ds(row_base, RPW)], len_v)
        for g in range(RPW // 16):
            lv = len_v[pl.ds(g * 16, 16)]
            recip_v[pl.ds(g * 16, 16)] = 1.0 / jnp.maximum(lv, 1).astype(
                jnp.float32)

        def start(p, rb, sem):
            pltpu.async_copy(table_h.at[idx_v.at[p]], rb, sem)

        def wait(p, rb, sem):
            pltpu.make_async_copy(table_h.at[idx_v.at[p]], rb, sem).wait()

        def process(p, rb):
            def lbody(l, accs):
                a0, a1, a2, a3, b0, b1, b2, b3 = accs
                return (
                    a0 + rb[l, pl.ds(0, 16)],
                    a1 + rb[l, pl.ds(16, 16)],
                    a2 + rb[l, pl.ds(32, 16)],
                    a3 + rb[l, pl.ds(48, 16)],
                    b0 + rb[l + L, pl.ds(0, 16)],
                    b1 + rb[l + L, pl.ds(16, 16)],
                    b2 + rb[l + L, pl.ds(32, 16)],
                    b3 + rb[l + L, pl.ds(48, 16)],
                )

            z = jnp.zeros((16,), jnp.float32)
            accs = lax.fori_loop(0, L, lbody, (z, z, z, z, z, z, z, z),
                                 unroll=10)
            j0 = 2 * p
            j1 = j0 + 1
            sv = recip_v[pl.ds(j0, 16)]
            s0 = sv[0]
            s1 = sv[1]
            out_v[j0, pl.ds(0, 16)] = accs[0] * s0
            out_v[j0, pl.ds(16, 16)] = accs[1] * s0
            out_v[j0, pl.ds(32, 16)] = accs[2] * s0
            out_v[j0, pl.ds(48, 16)] = accs[3] * s0
            out_v[j1, pl.ds(0, 16)] = accs[4] * s1
            out_v[j1, pl.ds(16, 16)] = accs[5] * s1
            out_v[j1, pl.ds(32, 16)] = accs[6] * s1
            out_v[j1, pl.ds(48, 16)] = accs[7] * s1

        for b in range(NB):
            start(b, rbs[b], sems[b])

        def step(s, carry):
            p0 = NB * s
            for b in range(NB):
                wait(p0 + b, rbs[b], sems[b])
                process(p0 + b, rbs[b])
                start(p0 + b + NB, rbs[b], sems[b])
            return carry

        lax.fori_loop(0, PAIRS // NB - 1, step, 0)
        for b in range(NB):
            p = PAIRS - NB + b
            wait(p, rbs[b], sems[b])
            process(p, rbs[b])

        pltpu.sync_copy(out_v, out_h.at[pl.ds(row_base, RPW)])

    return k


_sc_kernel = _build()


def kernel(keyword_lists, keyword_lengths, table):
    kw = (keyword_lists * 2).reshape(NW * PAIRS, GU)
    lens = keyword_lengths.reshape(B)
    tpad = jnp.pad(table, ((0, 0), (0, 64))).reshape(2 * V1, D)
    return _sc_kernel(kw, lens, tpad)
